# baseline (speedup 1.0000x reference)
import dataclasses
import functools

import jax
import jax.numpy as jnp
from jax import lax
from jax.experimental import pallas as pl
from jax.experimental.pallas import tpu as pltpu
from jax.experimental.pallas import tpu_sc as plsc

N = 2_000_000
TR = 16
TABLE_PAD = 10016

G = 625
R = 25
L = 128

T_0 = 227.13

_SIN_COEFS = (
    6.283185306817079, -41.34170217065687, 81.60524536016547,
    -76.70576094875487, 42.05737003862947, -15.084554589617913,
    3.775957048794309, -0.6150593859199129,
)
_K = 16.0


def _sc_model(table_pad, small_tab, tr3, pid3, day3, temp3, m3):
    mesh = plsc.VectorSubcoreMesh(core_axis_name="c", subcore_axis_name="s")
    cp = pltpu.CompilerParams()
    if "needs_layout_passes" in pltpu.CompilerParams.__dataclass_fields__:
        cp = dataclasses.replace(cp, needs_layout_passes=False)

    @functools.partial(
        pl.kernel,
        out_type=jax.ShapeDtypeStruct((G, R, L), jnp.float32),
        mesh=mesh,
        scratch_types=[
            pltpu.VMEM((TABLE_PAD,), jnp.float32),
            pltpu.VMEM((80,), jnp.float32),
        ],
        compiler_params=cp,
    )
    def sc_kernel(table_hbm, small_hbm, tr_hbm, pid_hbm, day_hbm, temp_hbm,
                  m_hbm, out_hbm, table_v, small_v):
        pltpu.sync_copy(table_hbm, table_v)
        pltpu.sync_copy(small_hbm, small_v)

        def body(tr_v, pid_v, day_v, temp_v, m_v, out_v):
            @plsc.parallel_loop(0, R, unroll=2)
            def _(r):
                for c in range(0, L, 16):
                    sl = pl.ds(c, 16)
                    tr = tr_v[0, r, sl] - 1
                    pid = pid_v[0, r, sl]
                    day = day_v[0, r, sl]
                    temp = temp_v[0, r, sl]
                    m = m_v[0, r, sl]

                    a_g = plsc.load_gather(small_v, [tr])
                    b_g = plsc.load_gather(small_v, [tr + 16])
                    nea_g = plsc.load_gather(small_v, [tr + 32])
                    amp_g = plsc.load_gather(small_v, [tr + 48])
                    pk_g = plsc.load_gather(small_v, [tr + 64])
                    a_big = plsc.load_gather(table_v, [pid])

                    xi_moist = a_g * m - b_g * (m * m)
                    xi_temp = a_big * jnp.exp(nea_g / (temp + (273.15 - T_0)))

                    y = day * (1.0 / 365.0) + pk_g
                    rf = lax.convert_element_type(
                        lax.convert_element_type(y, jnp.int32), jnp.float32
                    )
                    f1 = (y - rf) - 0.5
                    f = jnp.where(f1 < -0.5, f1 + 1.0, f1)
                    f2 = f * f
                    p = _SIN_COEFS[-1]
                    for coef in _SIN_COEFS[-2::-1]:
                        p = p * f2 + coef
                    out_v[0, r, sl] = amp_g * (f * p) + xi_temp * xi_moist

        blk = lambda: pl.BlockSpec((1, R, L), lambda i: (i, 0, 0))
        pltpu.emit_pipeline(
            body,
            grid=(G,),
            in_specs=[blk(), blk(), blk(), blk(), blk()],
            out_specs=[blk()],
            core_axis_name=("c", "s"),
            dimension_semantics=(pltpu.PARALLEL,),
        )(tr_hbm, pid_hbm, day_hbm, temp_hbm, m_hbm, out_hbm)

    return sc_kernel(table_pad, small_tab, tr3, pid3, day3, temp3, m3)


def kernel(treatment, plot_id, day_year, temp, resp, M,
           A, Ea, a, b, amplitude, peak_day):
    del resp
    table_pad = jnp.concatenate(
        [A[:1], A, jnp.zeros((TABLE_PAD - 10001,), jnp.float32)]
    )
    small_tab = jnp.concatenate([
        a, b, -Ea, amplitude, (peak_day - 1.0) * (1.0 / 365.0) + (_K + 0.5)
    ])

    shape3 = (G, R, L)
    out3 = _sc_model(
        table_pad,
        small_tab,
        treatment.reshape(shape3),
        plot_id.reshape(shape3),
        day_year.reshape(shape3),
        temp.reshape(shape3),
        M.reshape(shape3),
    )
    return out3.reshape(N)


# speedup vs baseline: 1.5108x; 1.5108x over previous
import dataclasses
import functools

import jax
import jax.numpy as jnp
from jax import lax
from jax.experimental import pallas as pl
from jax.experimental.pallas import tpu as pltpu
from jax.experimental.pallas import tpu_sc as plsc

N = 2_000_000
TR = 16
TABLE_PAD = 10016

SC_G = 125
SC_R = 125
SC_L = 128

G0 = 5
G1 = 3125
LANES = 128

T_0 = 227.13

_SIN_COEFS = (
    6.283185306817079, -41.34170217065687, 81.60524536016547,
    -76.70576094875487, 42.05737003862947, -15.084554589617913,
    3.775957048794309, -0.6150593859199129,
)

def _sc_gather_plot(table_pad, plot_id):
    mesh = plsc.VectorSubcoreMesh(core_axis_name="c", subcore_axis_name="s")
    cp = pltpu.CompilerParams()
    if "needs_layout_passes" in pltpu.CompilerParams.__dataclass_fields__:
        cp = dataclasses.replace(cp, needs_layout_passes=False)

    @functools.partial(
        pl.kernel,
        out_type=jax.ShapeDtypeStruct((SC_G, SC_R, SC_L), jnp.float32),
        mesh=mesh,
        scratch_types=[pltpu.VMEM((TABLE_PAD,), jnp.float32)],
        compiler_params=cp,
    )
    def sc_kernel(table_hbm, pid_hbm, out_hbm, table_v):
        pltpu.sync_copy(table_hbm, table_v)

        def body(idx_v, out_v):
            two_iota = lax.iota(jnp.int32, 16) * 2
            zeros = lax.iota(jnp.int32, 16) * 0

            @plsc.parallel_loop(0, SC_R, unroll=2)
            def _(r):
                for c in range(0, SC_L, 32):
                    iv_a = idx_v[0, r, pl.ds(c, 16)]
                    iv_b = idx_v[0, r, pl.ds(c + 16, 16)]
                    va = plsc.load_gather(table_v, [iv_a])
                    vb = plsc.load_gather(table_v, [iv_b])
                    out_v[0, r, pl.ds(c, 16)] = va
                    out_v[0, r, pl.ds(c + 16, 16)] = vb

        pltpu.emit_pipeline(
            body,
            grid=(SC_G,),
            in_specs=[pl.BlockSpec((1, SC_R, SC_L), lambda i: (i, 0, 0))],
            out_specs=[pl.BlockSpec((1, SC_R, SC_L), lambda i: (i, 0, 0))],
            core_axis_name=("c", "s"),
            dimension_semantics=(pltpu.PARALLEL,),
        )(pid_hbm, out_hbm)

    return sc_kernel(table_pad, plot_id)


def _tc_body(tr_ref, day_ref, temp_ref, m_ref, ag_ref,
             a_ref, b_ref, ea_ref, amp_ref, pk_ref, o_ref):
    idx = tr_ref[0].astype(jnp.int32) - 1

    def gather_tr(ref):
        tbl = jnp.broadcast_to(ref[...], (G1, LANES))
        return jnp.take_along_axis(tbl, idx, axis=1, mode="promise_in_bounds")

    a_g = gather_tr(a_ref)
    b_g = gather_tr(b_ref)
    ea_g = gather_tr(ea_ref)
    amp_g = gather_tr(amp_ref)
    pk_g = gather_tr(pk_ref)

    m = m_ref[0].astype(jnp.float32)
    temp = temp_ref[0].astype(jnp.float32)
    day = day_ref[0].astype(jnp.float32)
    a_big = ag_ref[0].astype(jnp.float32)

    xi_moist = a_g * m - b_g * (m * m)
    xi_temp = a_big * jnp.exp(-ea_g / (temp + 273.15 - T_0))
    t = (day + (pk_g - 1.0)) * (1.0 / 365.0)
    f = t - jnp.floor(t + 0.5)
    f2 = f * f
    p = _SIN_COEFS[-1]
    for c in _SIN_COEFS[-2::-1]:
        p = p * f2 + c
    sine_wave = amp_g * (f * p)
    o_ref[0] = sine_wave + xi_temp * xi_moist


def _tc_main(tr3, day3, temp3, m3, ag3, a_p, b_p, ea_p, amp_p, pk_p):
    blk3 = pl.BlockSpec((1, G1, LANES), lambda i: (i, 0, 0))
    blk_t = pl.BlockSpec((1, LANES), lambda i: (0, 0))
    return pl.pallas_call(
        _tc_body,
        grid=(G0,),
        in_specs=[blk3] * 5 + [blk_t] * 5,
        out_specs=blk3,
        out_shape=jax.ShapeDtypeStruct((G0, G1, LANES), jnp.float32),
    )(tr3, day3, temp3, m3, ag3, a_p, b_p, ea_p, amp_p, pk_p)


def kernel(treatment, plot_id, day_year, temp, resp, M,
           A, Ea, a, b, amplitude, peak_day):
    del resp
    table_pad = jnp.concatenate(
        [A[:1], A, jnp.zeros((TABLE_PAD - 10001,), jnp.float32)]
    )
    ag = _sc_gather_plot(table_pad, plot_id.reshape(SC_G, SC_R, SC_L))

    shape3 = (G0, G1, LANES)
    pad128 = lambda v: jnp.pad(v, (0, LANES - TR)).reshape(1, LANES)
    out3 = _tc_main(
        treatment.reshape(shape3),
        day_year.reshape(shape3),
        temp.reshape(shape3),
        M.reshape(shape3),
        ag.reshape(shape3),
        pad128(a),
        pad128(b),
        pad128(Ea),
        pad128(amplitude),
        pad128(peak_day),
    )
    return out3.reshape(N)
